# speedup vs baseline: 1.4333x; 1.4333x over previous
import jax
import jax.numpy as jnp
from jax.experimental import pallas as pl

_C = 768
_H = 12
_HD = 64
_BQ = 1024


def _attn_body(x_ref, wq_ref, wk_ref, wv_ref, bq_ref, bk_ref, bv_ref, ao_ref):
    xb = x_ref[0]
    n = xb.shape[0]
    q = jnp.dot(xb, wq_ref[0], preferred_element_type=jnp.float32) + bq_ref[0, 0]
    k = jnp.dot(xb, wk_ref[0], preferred_element_type=jnp.float32) + bk_ref[0, 0]
    v = jnp.dot(xb, wv_ref[0], preferred_element_type=jnp.float32) + bv_ref[0, 0]
    q = q * (_HD ** -0.5)
    for i in range(n // _BQ):
        qi = q[i * _BQ:(i + 1) * _BQ]
        s = jax.lax.dot_general(qi, k, (((1,), (1,)), ((), ())),
                                preferred_element_type=jnp.float32)
        m = jnp.max(s, axis=-1, keepdims=True)
        e = jnp.exp(s - m)
        p = e / jnp.sum(e, axis=-1, keepdims=True)
        ao_ref[0, 0, i * _BQ:(i + 1) * _BQ, :] = jnp.dot(
            p, v, preferred_element_type=jnp.float32)


def _proj_body(ao_ref, pw_ref, pb_ref, out_ref):
    h = pl.program_id(1)
    contrib = jnp.dot(ao_ref[0, 0], pw_ref[0], preferred_element_type=jnp.float32)

    @pl.when(h == 0)
    def _():
        out_ref[0] = contrib + pb_ref[0]

    @pl.when(h != 0)
    def _():
        out_ref[0] = out_ref[0] + contrib


def kernel(x, xpos, qkv_w, qkv_b, proj_w, proj_b):
    del xpos
    B, N, C = x.shape
    w3 = jnp.transpose(qkv_w.reshape(3 * _H, _HD, C), (0, 2, 1))
    b3 = qkv_b.reshape(3 * _H, 1, _HD)
    pw3 = jnp.transpose(proj_w, (1, 0)).reshape(_H, _HD, C)
    pb2 = proj_b.reshape(1, C)

    attnout = pl.pallas_call(
        _attn_body,
        grid=(B, _H),
        in_specs=[
            pl.BlockSpec((1, N, C), lambda b, h: (b, 0, 0)),
            pl.BlockSpec((1, C, _HD), lambda b, h: (h, 0, 0)),
            pl.BlockSpec((1, C, _HD), lambda b, h: (_H + h, 0, 0)),
            pl.BlockSpec((1, C, _HD), lambda b, h: (2 * _H + h, 0, 0)),
            pl.BlockSpec((1, 1, _HD), lambda b, h: (h, 0, 0)),
            pl.BlockSpec((1, 1, _HD), lambda b, h: (_H + h, 0, 0)),
            pl.BlockSpec((1, 1, _HD), lambda b, h: (2 * _H + h, 0, 0)),
        ],
        out_specs=pl.BlockSpec((1, 1, N, _HD), lambda b, h: (b, h, 0, 0)),
        out_shape=jax.ShapeDtypeStruct((B, _H, N, _HD), jnp.float32),
    )(x, w3, w3, w3, b3, b3, b3)

    out = pl.pallas_call(
        _proj_body,
        grid=(B, _H),
        in_specs=[
            pl.BlockSpec((1, 1, N, _HD), lambda b, h: (b, h, 0, 0)),
            pl.BlockSpec((1, _HD, C), lambda b, h: (h, 0, 0)),
            pl.BlockSpec((1, C), lambda b, h: (0, 0)),
        ],
        out_specs=pl.BlockSpec((1, N, C), lambda b, h: (b, 0, 0)),
        out_shape=jax.ShapeDtypeStruct((B, N, C), jnp.float32),
    )(attnout, pw3, pb2)
    return out


# speedup vs baseline: 1.8708x; 1.3052x over previous
import jax
import jax.numpy as jnp
from jax.experimental import pallas as pl

_C = 768
_H = 12
_HD = 64
_BQ = 1024


def _attn_body(x_ref, w_ref, b_ref, ao_ref):
    xb = x_ref[0]
    n = xb.shape[0]
    qkv = jnp.dot(xb, w_ref[0], preferred_element_type=jnp.float32) + b_ref[0, 0]
    q = qkv[:, 0:_HD] * (_HD ** -0.5)
    k = qkv[:, _HD:2 * _HD]
    v = qkv[:, 2 * _HD:3 * _HD]
    for i in range(n // _BQ):
        qi = q[i * _BQ:(i + 1) * _BQ]
        s = jax.lax.dot_general(qi, k, (((1,), (1,)), ((), ())),
                                preferred_element_type=jnp.float32)
        m = jnp.max(s, axis=-1, keepdims=True)
        e = jnp.exp(s - m)
        d = jnp.sum(e, axis=-1, keepdims=True)
        o = jnp.dot(e, v, preferred_element_type=jnp.float32)
        ao_ref[0, 0, i * _BQ:(i + 1) * _BQ, :] = o * (1.0 / d)


def _proj_body(ao_ref, pw_ref, pb_ref, out_ref):
    acc = pb_ref[0] + jnp.zeros((ao_ref.shape[2], ao_ref.shape[3] * _H),
                                jnp.float32)
    for h in range(_H):
        acc = acc + jnp.dot(ao_ref[0, h], pw_ref[h],
                            preferred_element_type=jnp.float32)
    out_ref[0] = acc


def kernel(x, xpos, qkv_w, qkv_b, proj_w, proj_b):
    del xpos
    B, N, C = x.shape
    w192 = jnp.transpose(qkv_w.reshape(3, _H, _HD, C), (1, 3, 0, 2)).reshape(_H, C, 3 * _HD)
    b192 = jnp.transpose(qkv_b.reshape(3, _H, _HD), (1, 0, 2)).reshape(_H, 1, 3 * _HD)
    pw3 = jnp.transpose(proj_w, (1, 0)).reshape(_H, _HD, C)
    pb2 = proj_b.reshape(1, C)

    attnout = pl.pallas_call(
        _attn_body,
        grid=(B, _H),
        in_specs=[
            pl.BlockSpec((1, N, C), lambda b, h: (b, 0, 0)),
            pl.BlockSpec((1, C, 3 * _HD), lambda b, h: (h, 0, 0)),
            pl.BlockSpec((1, 1, 3 * _HD), lambda b, h: (h, 0, 0)),
        ],
        out_specs=pl.BlockSpec((1, 1, N, _HD), lambda b, h: (b, h, 0, 0)),
        out_shape=jax.ShapeDtypeStruct((B, _H, N, _HD), jnp.float32),
    )(x, w192, b192)

    out = pl.pallas_call(
        _proj_body,
        grid=(B,),
        in_specs=[
            pl.BlockSpec((1, _H, N, _HD), lambda b: (b, 0, 0, 0)),
            pl.BlockSpec((_H, _HD, C), lambda b: (0, 0, 0)),
            pl.BlockSpec((1, C), lambda b: (0, 0)),
        ],
        out_specs=pl.BlockSpec((1, N, C), lambda b: (b, 0, 0)),
        out_shape=jax.ShapeDtypeStruct((B, N, C), jnp.float32),
    )(attnout, pw3, pb2)
    return out


# speedup vs baseline: 2.0196x; 1.0795x over previous
import jax
import jax.numpy as jnp
from jax.experimental import pallas as pl

_C = 768
_H = 12
_HD = 64
_BQ = 1024


def _attn_body(x_ref, w_ref, b_ref, ao_ref):
    xb = x_ref[0]
    n = xb.shape[0]
    qkv = jnp.dot(xb, w_ref[0], preferred_element_type=jnp.float32) + b_ref[0, 0]
    q = qkv[:, 0:_HD] * (_HD ** -0.5)
    k = qkv[:, _HD:2 * _HD]
    v = qkv[:, 2 * _HD:3 * _HD]
    v_ext = jnp.concatenate([v, jnp.ones((n, 1), jnp.float32)], axis=-1)
    for i in range(n // _BQ):
        qi = q[i * _BQ:(i + 1) * _BQ]
        s = jax.lax.dot_general(qi, k, (((1,), (1,)), ((), ())),
                                preferred_element_type=jnp.float32)
        m = jnp.max(s, axis=-1, keepdims=True)
        e = jnp.exp(s - m)
        o = jnp.dot(e, v_ext, preferred_element_type=jnp.float32)
        ao_ref[0, 0, i * _BQ:(i + 1) * _BQ, :] = (
            o[:, :_HD] * (1.0 / o[:, _HD:_HD + 1]))


def _proj_body(ao_ref, pw_ref, pb_ref, out_ref):
    acc = pb_ref[0] + jnp.zeros((ao_ref.shape[2], ao_ref.shape[3] * _H),
                                jnp.float32)
    for h in range(_H):
        acc = acc + jnp.dot(ao_ref[0, h], pw_ref[h],
                            preferred_element_type=jnp.float32)
    out_ref[0] = acc


def kernel(x, xpos, qkv_w, qkv_b, proj_w, proj_b):
    del xpos
    B, N, C = x.shape
    w192 = jnp.transpose(qkv_w.reshape(3, _H, _HD, C), (1, 3, 0, 2)).reshape(_H, C, 3 * _HD)
    b192 = jnp.transpose(qkv_b.reshape(3, _H, _HD), (1, 0, 2)).reshape(_H, 1, 3 * _HD)
    pw3 = jnp.transpose(proj_w, (1, 0)).reshape(_H, _HD, C)
    pb2 = proj_b.reshape(1, C)

    attnout = pl.pallas_call(
        _attn_body,
        grid=(B, _H),
        in_specs=[
            pl.BlockSpec((1, N, C), lambda b, h: (b, 0, 0)),
            pl.BlockSpec((1, C, 3 * _HD), lambda b, h: (h, 0, 0)),
            pl.BlockSpec((1, 1, 3 * _HD), lambda b, h: (h, 0, 0)),
        ],
        out_specs=pl.BlockSpec((1, 1, N, _HD), lambda b, h: (b, h, 0, 0)),
        out_shape=jax.ShapeDtypeStruct((B, _H, N, _HD), jnp.float32),
    )(x, w192, b192)

    out = pl.pallas_call(
        _proj_body,
        grid=(B,),
        in_specs=[
            pl.BlockSpec((1, _H, N, _HD), lambda b: (b, 0, 0, 0)),
            pl.BlockSpec((_H, _HD, C), lambda b: (0, 0, 0)),
            pl.BlockSpec((1, C), lambda b: (0, 0)),
        ],
        out_specs=pl.BlockSpec((1, N, C), lambda b: (b, 0, 0)),
        out_shape=jax.ShapeDtypeStruct((B, N, C), jnp.float32),
    )(attnout, pw3, pb2)
    return out
